# baseline (speedup 1.0000x reference)
import functools

import jax
import jax.numpy as jnp
from jax import lax
from jax.experimental import pallas as pl
from jax.experimental.pallas import tpu as pltpu
from jax.experimental.pallas import tpu_sc as plsc

_NODE_NUM = 1000000
_TAG_NUM = 100000
_EMB_NUM = 4
_D = 32
_B = 4096
_TOT = 2 * _B
_LANES = 128


def _sc_gather(flat_nodes, tag_table, nidx4, tidx):
    info = plsc.get_sparse_core_info()
    nw = info.num_cores * info.num_subcores
    per = _TOT // nw
    chunks = per // _LANES
    mesh = plsc.VectorSubcoreMesh(core_axis_name="c", subcore_axis_name="s")

    @functools.partial(
        pl.kernel,
        mesh=mesh,
        compiler_params=pltpu.CompilerParams(use_tc_tiling_on_sc=False),
        out_type=[
            jax.ShapeDtypeStruct((_EMB_NUM, _TOT, _D), jnp.float32),
            jax.ShapeDtypeStruct((_TOT, _D), jnp.float32),
        ],
        scratch_types=[
            pltpu.VMEM((_EMB_NUM, chunks, _LANES), jnp.int32),
            pltpu.VMEM((chunks, _LANES), jnp.int32),
            pltpu.VMEM((_EMB_NUM, per, _D), jnp.float32),
            pltpu.VMEM((per, _D), jnp.float32),
            pltpu.SemaphoreType.DMA,
        ],
    )
    def k(nodes_hbm, tags_hbm, nidx_hbm, tidx_hbm, nrows_out, trows_out,
          nidx_v, tidx_v, nrows_v, trows_v, sem):
        wid = lax.axis_index("s") * info.num_cores + lax.axis_index("c")
        base = wid * per
        crow = wid * chunks
        for i in range(_EMB_NUM):
            pltpu.sync_copy(nidx_hbm.at[i, pl.ds(crow, chunks)], nidx_v.at[i])
        pltpu.sync_copy(tidx_hbm.at[pl.ds(crow, chunks)], tidx_v)
        cps = []
        for i in range(_EMB_NUM):
            for j in range(chunks):
                cps.append(pltpu.async_copy(
                    nodes_hbm.at[nidx_v.at[i, j]],
                    nrows_v.at[i, pl.ds(j * _LANES, _LANES)], sem))
        for j in range(chunks):
            cps.append(pltpu.async_copy(
                tags_hbm.at[tidx_v.at[j]],
                trows_v.at[pl.ds(j * _LANES, _LANES)], sem))
        for cp in cps:
            cp.wait()
        for i in range(_EMB_NUM):
            pltpu.sync_copy(nrows_v.at[i], nrows_out.at[i, pl.ds(base, per)])
        pltpu.sync_copy(trows_v, trows_out.at[pl.ds(base, per)])

    return k(flat_nodes, tag_table, nidx4, tidx)


def _loss_body(nr_ref, tr_ref, o_ref):
    tr = tr_ref[...]
    m = None
    for i in range(_EMB_NUM):
        d = jnp.sum(nr_ref[i] * tr, axis=-1, keepdims=True)
        m = d if m is None else jnp.maximum(m, d)
    row = lax.broadcasted_iota(jnp.int32, (_TOT, 1), 0)
    x = jnp.where(row < _B, -m, m)
    sp = jnp.maximum(x, 0.0) + jnp.log(1.0 + jnp.exp(-jnp.abs(x)))
    o_ref[...] = jnp.sum(sp).reshape(1, 1)


def _tc_loss(node_rows, tag_rows):
    return pl.pallas_call(
        _loss_body,
        out_shape=jax.ShapeDtypeStruct((1, 1), jnp.float32),
    )(node_rows, tag_rows)


def kernel(node_embs, tag_table, pos_node, pos_tag, neg_node, neg_tag):
    flat_nodes = node_embs.reshape(_EMB_NUM * _NODE_NUM, _D)
    node_idx = jnp.concatenate([pos_node, neg_node])
    tag_idx = jnp.concatenate([pos_tag, neg_tag])
    offs = (jnp.arange(_EMB_NUM, dtype=jnp.int32) * _NODE_NUM)[:, None]
    nidx4 = (node_idx[None, :] + offs).reshape(_EMB_NUM, _TOT // _LANES, _LANES)
    tidx = tag_idx.reshape(_TOT // _LANES, _LANES)
    node_rows, tag_rows = _sc_gather(flat_nodes, tag_table, nidx4, tidx)
    out = _tc_loss(node_rows, tag_rows)
    return out[0, 0]


# speedup vs baseline: 5.9794x; 5.9794x over previous
import functools

import jax
import jax.numpy as jnp
from jax import lax
from jax.experimental import pallas as pl
from jax.experimental.pallas import tpu as pltpu
from jax.experimental.pallas import tpu_sc as plsc

_NODE_NUM = 1000000
_TAG_NUM = 100000
_EMB_NUM = 4
_D = 32
_B = 4096
_TOT = 2 * _B
_R = _EMB_NUM * _D
_LAST_NC = (_NODE_NUM - 1) // 128
_LAST_TC = (_TAG_NUM - 1) // 128


def _sc_gather(tb, ntail, tg, ttail, nidx, tidx):
    info = plsc.get_sparse_core_info()
    nc = info.num_cores
    nw = nc * info.num_subcores
    per = _TOT // nw
    groups = per // 16
    mesh = plsc.VectorSubcoreMesh(core_axis_name="c", subcore_axis_name="s")

    @functools.partial(
        pl.kernel,
        mesh=mesh,
        compiler_params=pltpu.CompilerParams(
            use_tc_tiling_on_sc=True, needs_layout_passes=False),
        out_type=[
            jax.ShapeDtypeStruct((_TOT, 128), jnp.float32),
            jax.ShapeDtypeStruct((_TOT, 128), jnp.float32),
        ],
        scratch_types=[
            pltpu.VMEM((per,), jnp.int32),
            pltpu.VMEM((per,), jnp.int32),
            pltpu.VMEM((2, _R, 128), jnp.float32),
            pltpu.VMEM((2, _D, 128), jnp.float32),
            pltpu.VMEM((16, 128), jnp.float32),
            pltpu.VMEM((16, 128), jnp.float32),
            pltpu.SemaphoreType.DMA,
            pltpu.SemaphoreType.DMA,
            pltpu.SemaphoreType.DMA,
            pltpu.SemaphoreType.DMA,
            pltpu.SemaphoreType.DMA,
        ],
    )
    def k(tb_h, ntail_h, tg_h, ttail_h, nidx_h, tidx_h, g_out, t_out,
          nidx_v, tidx_v, nblk, tblk, nsb, tsb, nsem0, nsem1, tsem0, tsem1,
          ssem):
        wid = lax.axis_index("s") * nc + lax.axis_index("c")
        base = wid * per
        pltpu.sync_copy(nidx_h.at[pl.ds(base, per)], nidx_v)
        pltpu.sync_copy(tidx_h.at[pl.ds(base, per)], tidx_v)
        nsems = (nsem0, nsem1)
        tsems = (tsem0, tsem1)

        def fire(n, t, p):
            ncol = n >> 7
            noff = pl.multiple_of(jnp.minimum(ncol, _LAST_NC) * 128, 128)

            @pl.when(ncol < _LAST_NC)
            def _():
                pltpu.async_copy(
                    tb_h.at[:, pl.ds(noff, 128)], nblk.at[p], nsems[p])

            @pl.when(ncol >= _LAST_NC)
            def _():
                pltpu.async_copy(ntail_h.at[:, :], nblk.at[p], nsems[p])

            tcol = t >> 7
            toff = pl.multiple_of(jnp.minimum(tcol, _LAST_TC) * 128, 128)

            @pl.when(tcol < _LAST_TC)
            def _():
                pltpu.async_copy(
                    tg_h.at[:, pl.ds(toff, 128)], tblk.at[p], tsems[p])

            @pl.when(tcol >= _LAST_TC)
            def _():
                pltpu.async_copy(ttail_h.at[:, :], tblk.at[p], tsems[p])

        def drain(p):
            pltpu.make_async_copy(
                ntail_h.at[:, :], nblk.at[p], nsems[p]).wait()
            pltpu.make_async_copy(
                ttail_h.at[:, :], tblk.at[p], tsems[p]).wait()

        head = nidx_v[pl.ds(0, 16)]
        thead = tidx_v[pl.ds(0, 16)]
        fire(head[0], thead[0], 0)
        fire(head[1], thead[1], 1)

        def group(o, _):
            nvec = nidx_v[pl.ds(o * 16, 16)]
            tvec = tidx_v[pl.ds(o * 16, 16)]
            nxt = (o + 1) * 16 & (per - 1)
            nvec2 = nidx_v[pl.ds(nxt, 16)]
            tvec2 = tidx_v[pl.ds(nxt, 16)]
            for i in range(16):
                p = i & 1
                drain(p)
                nlane = jnp.full((16,), nvec[i] & 127, jnp.int32)
                for kk in range(_R // 16):
                    rows = lax.iota(jnp.int32, 16) + kk * 16
                    nsb[i, pl.ds(kk * 16, 16)] = plsc.load_gather(
                        nblk.at[p], [rows, nlane])
                tlane = jnp.full((16,), tvec[i] & 127, jnp.int32)
                for kk in range(_D // 16):
                    rows = lax.iota(jnp.int32, 16) + kk * 16
                    tsb[i, pl.ds(kk * 16, 16)] = plsc.load_gather(
                        tblk.at[p], [rows, tlane])
                if i < 14:
                    fire(nvec[i + 2], tvec[i + 2], p)
                else:

                    @pl.when(o < groups - 1)
                    def _():
                        fire(nvec2[i - 14], tvec2[i - 14], p)

            jvec = base + o * 16 + lax.iota(jnp.int32, 16)
            cp1 = pltpu.async_copy(nsb, g_out.at[jvec], ssem)
            cp2 = pltpu.async_copy(tsb, t_out.at[jvec], ssem)
            cp1.wait()
            cp2.wait()
            return _

        lax.fori_loop(0, groups, group, None)

    return k(tb, ntail, tg, ttail, nidx, tidx)


def _loss_body(g_ref, t_ref, o_ref):
    tg = t_ref[:, 0:_D]
    m = None
    for e in range(_EMB_NUM):
        d = jnp.sum(g_ref[:, e * _D:(e + 1) * _D] * tg, axis=-1,
                    keepdims=True)
        m = d if m is None else jnp.maximum(m, d)
    row = lax.broadcasted_iota(jnp.int32, (_TOT, 1), 0)
    x = jnp.where(row < _B, -m, m)
    sp = jnp.maximum(x, 0.0) + jnp.log(1.0 + jnp.exp(-jnp.abs(x)))
    o_ref[...] = jnp.sum(sp).reshape(1, 1)


def _tc_loss(g, tg):
    return pl.pallas_call(
        _loss_body,
        out_shape=jax.ShapeDtypeStruct((1, 1), jnp.float32),
    )(g, tg)


def kernel(node_embs, tag_table, pos_node, pos_tag, neg_node, neg_tag):
    tb = jnp.transpose(node_embs, (0, 2, 1)).reshape(_R, _NODE_NUM)
    tg = jnp.transpose(tag_table, (1, 0))
    ntail = jnp.pad(tb[:, _LAST_NC * 128:],
                    ((0, 0), (0, 128 - (_NODE_NUM - _LAST_NC * 128))))
    ttail = jnp.pad(tg[:, _LAST_TC * 128:],
                    ((0, 0), (0, 128 - (_TAG_NUM - _LAST_TC * 128))))
    nidx = jnp.concatenate([pos_node, neg_node])
    tidx = jnp.concatenate([pos_tag, neg_tag])
    g, tgr = _sc_gather(tb, ntail, tg, ttail, nidx, tidx)
    return _tc_loss(g, tgr)[0, 0]


# speedup vs baseline: 7.0834x; 1.1846x over previous
import functools

import jax
import jax.numpy as jnp
from jax import lax
from jax.experimental import pallas as pl
from jax.experimental.pallas import tpu as pltpu
from jax.experimental.pallas import tpu_sc as plsc

_NODE_NUM = 1000000
_TAG_NUM = 100000
_EMB_NUM = 4
_D = 32
_B = 4096
_TOT = 2 * _B
_R = _EMB_NUM * _D
_LAST_NC = (_NODE_NUM - 1) // 128
_LAST_TC = (_TAG_NUM - 1) // 128


def _sc_gather(tb, ntail, tg, ttail, nidx, tidx):
    info = plsc.get_sparse_core_info()
    nc = info.num_cores
    nw = nc * info.num_subcores
    per = _TOT // nw
    groups = per // 16
    mesh = plsc.VectorSubcoreMesh(core_axis_name="c", subcore_axis_name="s")

    @functools.partial(
        pl.kernel,
        mesh=mesh,
        compiler_params=pltpu.CompilerParams(
            use_tc_tiling_on_sc=True, needs_layout_passes=False),
        out_type=[
            jax.ShapeDtypeStruct((_TOT, 128), jnp.float32),
            jax.ShapeDtypeStruct((_TOT, 128), jnp.float32),
        ],
        scratch_types=[
            pltpu.VMEM((per,), jnp.int32),
            pltpu.VMEM((per,), jnp.int32),
            pltpu.VMEM((4, _R, 128), jnp.float32),
            pltpu.VMEM((4, _D, 128), jnp.float32),
            pltpu.VMEM((16, 128), jnp.float32),
            pltpu.VMEM((16, 128), jnp.float32),
            pltpu.SemaphoreType.DMA,
            pltpu.SemaphoreType.DMA,
            pltpu.SemaphoreType.DMA,
            pltpu.SemaphoreType.DMA,
            pltpu.SemaphoreType.DMA,
            pltpu.SemaphoreType.DMA,
            pltpu.SemaphoreType.DMA,
            pltpu.SemaphoreType.DMA,
            pltpu.SemaphoreType.DMA,
        ],
    )
    def k(tb_h, ntail_h, tg_h, ttail_h, nidx_h, tidx_h, g_out, t_out,
          nidx_v, tidx_v, nblk, tblk, nsb, tsb, nsem0, nsem1, nsem2, nsem3,
          tsem0, tsem1, tsem2, tsem3, ssem):
        wid = lax.axis_index("s") * nc + lax.axis_index("c")
        base = wid * per
        pltpu.sync_copy(nidx_h.at[pl.ds(base, per)], nidx_v)
        pltpu.sync_copy(tidx_h.at[pl.ds(base, per)], tidx_v)
        nsems = (nsem0, nsem1, nsem2, nsem3)
        tsems = (tsem0, tsem1, tsem2, tsem3)

        def fire(n, t, p):
            ncol = n >> 7
            noff = pl.multiple_of(jnp.minimum(ncol, _LAST_NC) * 128, 128)

            @pl.when(ncol < _LAST_NC)
            def _():
                pltpu.async_copy(
                    tb_h.at[:, pl.ds(noff, 128)], nblk.at[p], nsems[p])

            @pl.when(ncol >= _LAST_NC)
            def _():
                pltpu.async_copy(ntail_h.at[:, :], nblk.at[p], nsems[p])

            tcol = t >> 7
            toff = pl.multiple_of(jnp.minimum(tcol, _LAST_TC) * 128, 128)

            @pl.when(tcol < _LAST_TC)
            def _():
                pltpu.async_copy(
                    tg_h.at[:, pl.ds(toff, 128)], tblk.at[p], tsems[p])

            @pl.when(tcol >= _LAST_TC)
            def _():
                pltpu.async_copy(ttail_h.at[:, :], tblk.at[p], tsems[p])

        def drain(p):
            pltpu.make_async_copy(
                ntail_h.at[:, :], nblk.at[p], nsems[p]).wait()
            pltpu.make_async_copy(
                ttail_h.at[:, :], tblk.at[p], tsems[p]).wait()

        head = nidx_v[pl.ds(0, 16)]
        thead = tidx_v[pl.ds(0, 16)]
        for p in range(4):
            fire(head[p], thead[p], p)

        def group(o, _):
            nvec = nidx_v[pl.ds(o * 16, 16)]
            tvec = tidx_v[pl.ds(o * 16, 16)]
            nxt = (o + 1) * 16 & (per - 1)
            nvec2 = nidx_v[pl.ds(nxt, 16)]
            tvec2 = tidx_v[pl.ds(nxt, 16)]
            for i in range(16):
                p = i & 3
                drain(p)
                nlane = jnp.full((16,), nvec[i] & 127, jnp.int32)
                for kk in range(_R // 16):
                    rows = lax.iota(jnp.int32, 16) + kk * 16
                    nsb[i, pl.ds(kk * 16, 16)] = plsc.load_gather(
                        nblk.at[p], [rows, nlane])
                tlane = jnp.full((16,), tvec[i] & 127, jnp.int32)
                for kk in range(_D // 16):
                    rows = lax.iota(jnp.int32, 16) + kk * 16
                    tsb[i, pl.ds(kk * 16, 16)] = plsc.load_gather(
                        tblk.at[p], [rows, tlane])
                if i < 12:
                    fire(nvec[i + 4], tvec[i + 4], p)
                else:

                    @pl.when(o < groups - 1)
                    def _():
                        fire(nvec2[i - 12], tvec2[i - 12], p)

            jvec = base + o * 16 + lax.iota(jnp.int32, 16)
            cp1 = pltpu.async_copy(nsb, g_out.at[jvec], ssem)
            cp2 = pltpu.async_copy(tsb, t_out.at[jvec], ssem)
            cp1.wait()
            cp2.wait()
            return _

        lax.fori_loop(0, groups, group, None)

    return k(tb, ntail, tg, ttail, nidx, tidx)


def _loss_body(g_ref, t_ref, o_ref):
    tg = t_ref[:, 0:_D]
    m = None
    for e in range(_EMB_NUM):
        d = jnp.sum(g_ref[:, e * _D:(e + 1) * _D] * tg, axis=-1,
                    keepdims=True)
        m = d if m is None else jnp.maximum(m, d)
    row = lax.broadcasted_iota(jnp.int32, (_TOT, 1), 0)
    x = jnp.where(row < _B, -m, m)
    sp = jnp.maximum(x, 0.0) + jnp.log(1.0 + jnp.exp(-jnp.abs(x)))
    o_ref[...] = jnp.sum(sp).reshape(1, 1)


def _tc_loss(g, tg):
    return pl.pallas_call(
        _loss_body,
        out_shape=jax.ShapeDtypeStruct((1, 1), jnp.float32),
    )(g, tg)


def kernel(node_embs, tag_table, pos_node, pos_tag, neg_node, neg_tag):
    tb = jnp.transpose(node_embs, (0, 2, 1)).reshape(_R, _NODE_NUM)
    tg = jnp.transpose(tag_table, (1, 0))
    ntail = jnp.pad(tb[:, _LAST_NC * 128:],
                    ((0, 0), (0, 128 - (_NODE_NUM - _LAST_NC * 128))))
    ttail = jnp.pad(tg[:, _LAST_TC * 128:],
                    ((0, 0), (0, 128 - (_TAG_NUM - _LAST_TC * 128))))
    nidx = jnp.concatenate([pos_node, neg_node])
    tidx = jnp.concatenate([pos_tag, neg_tag])
    g, tgr = _sc_gather(tb, ntail, tg, ttail, nidx, tidx)
    return _tc_loss(g, tgr)[0, 0]
